# baseline (speedup 1.0000x reference)
import jax
import jax.numpy as jnp
from jax.experimental import pallas as pl
from jax.experimental.pallas import tpu as pltpu

_TB = 256
_CH = 256


def _rvq_body(xt_ref, e_ref, agg_ref, ind_ref, es_ref, e2_ref):
    tb, d = xt_ref.shape
    c_num, k, _ = e_ref.shape
    h = tb // 2
    nch = k // _CH

    @pl.when(pl.program_id(0) == 0)
    def _():
        for c in range(c_num):
            e = e_ref[c, :, :d]
            e2col = jnp.sum(e * e, axis=1, keepdims=True)
            e2_ref[c:c + 1, :] = jnp.transpose(e2col, (1, 0))
            hi = e.astype(jnp.bfloat16)
            r1 = e - hi.astype(jnp.float32)
            mid = (r1 * (2.0 ** 9)).astype(jnp.bfloat16)
            r2 = r1 - mid.astype(jnp.float32) * (2.0 ** -9)
            lo = (r2 * (2.0 ** 18)).astype(jnp.bfloat16)
            r3 = r2 - lo.astype(jnp.float32) * (2.0 ** -18)
            lo2 = (r3 * (2.0 ** 27)).astype(jnp.bfloat16)
            es_ref[c, :, 0 * d:1 * d] = hi
            es_ref[c, :, 1 * d:2 * d] = mid
            es_ref[c, :, 2 * d:3 * d] = lo
            es_ref[c, :, 3 * d:4 * d] = lo2

    iota_f = jax.lax.broadcasted_iota(jnp.int32, (h, k), 1).astype(jnp.float32)
    xs = [xt_ref[:h], xt_ref[h:]]
    zqs = [jnp.zeros((h, d), jnp.float32) for _ in range(2)]
    for c in range(c_num):
        es = es_ref[c]
        for j in range(2):
            x_res = xs[j]
            x2 = jnp.sum(x_res * x_res, axis=1, keepdims=True)
            xaug = jnp.concatenate([x_res * -2.0, x2], axis=1)
            q = jax.lax.dot_general(
                xaug, e_ref[c],
                (((1,), (1,)), ((), ())))
            m_run = jnp.full((h, 1), jnp.inf, jnp.float32)
            i_run = jnp.full((h, 1), float(k), jnp.float32)
            for cc in range(nch):
                lo_, hi_ = cc * _CH, (cc + 1) * _CH
                t = q[:, lo_:hi_] + e2_ref[c:c + 1, lo_:hi_]
                mc = jnp.min(t, axis=1, keepdims=True)
                ic = jnp.min(jnp.where(t == mc, iota_f[:, lo_:hi_], float(k)),
                             axis=1, keepdims=True)
                first = mc < m_run
                i_run = jnp.where(first, ic, i_run)
                m_run = jnp.minimum(mc, m_run)
            indf = i_run
            oh = (iota_f == indf).astype(jnp.bfloat16)
            parts = jax.lax.dot_general(
                oh, es, (((1,), (0,)), ((), ())),
                preferred_element_type=jnp.float32)
            sel = ((parts[:, :d] + parts[:, d:2 * d] * (2.0 ** -9))
                   + parts[:, 2 * d:3 * d] * (2.0 ** -18)) \
                + parts[:, 3 * d:] * (2.0 ** -27)
            xs[j] = x_res - sel
            zqs[j] = zqs[j] + sel
            agg_ref[c, j * h:(j + 1) * h] = zqs[j]
            ind_ref[c, j * h:(j + 1) * h] = indf[:, 0].astype(jnp.int32)


def kernel(x_in, code_embeddings):
    b, d, t = x_in.shape
    c_num, k, _ = code_embeddings.shape
    nt = b * t
    xt = jnp.transpose(x_in, (0, 2, 1)).reshape(nt, d)
    e_aug = jnp.concatenate(
        [code_embeddings, jnp.ones((c_num, k, 1), jnp.float32)], axis=-1)
    grid = (pl.cdiv(nt, _TB),)
    aggs, inds = pl.pallas_call(
        _rvq_body,
        grid=grid,
        in_specs=[
            pl.BlockSpec((_TB, d), lambda i: (i, 0)),
            pl.BlockSpec((c_num, k, d + 1), lambda i: (0, 0, 0)),
        ],
        out_specs=[
            pl.BlockSpec((c_num, _TB, d), lambda i: (0, i, 0)),
            pl.BlockSpec((c_num, _TB), lambda i: (0, i)),
        ],
        out_shape=[
            jax.ShapeDtypeStruct((c_num, nt, d), jnp.float32),
            jax.ShapeDtypeStruct((c_num, nt), jnp.int32),
        ],
        scratch_shapes=[
            pltpu.VMEM((c_num, k, 4 * d), jnp.bfloat16),
            pltpu.VMEM((c_num, k), jnp.float32),
        ],
    )(xt, e_aug)
    z_q_aggregated = jnp.transpose(aggs.reshape(c_num, b, t, d), (1, 0, 3, 2))
    indices = jnp.transpose(inds.reshape(c_num, b, t), (1, 2, 0))
    return z_q_aggregated, indices


# speedup vs baseline: 1.0753x; 1.0753x over previous
import jax
import jax.numpy as jnp
from jax.experimental import pallas as pl
from jax.experimental.pallas import tpu as pltpu

_TB = 256
_CH = 256


def _rvq_body(xt_ref, e_ref, agg_ref, ind_ref, es_ref, e2_ref):
    tb, d = xt_ref.shape
    c_num, k, _ = e_ref.shape
    h = tb // 2
    nch = k // _CH

    @pl.when(pl.program_id(0) == 0)
    def _():
        for c in range(c_num):
            e = e_ref[c]
            e2col = jnp.sum(e * e, axis=1, keepdims=True)
            e2_ref[c:c + 1, :] = jnp.transpose(e2col, (1, 0))
            hi = e.astype(jnp.bfloat16)
            r1 = e - hi.astype(jnp.float32)
            mid = (r1 * (2.0 ** 9)).astype(jnp.bfloat16)
            r2 = r1 - mid.astype(jnp.float32) * (2.0 ** -9)
            lo = (r2 * (2.0 ** 18)).astype(jnp.bfloat16)
            r3 = r2 - lo.astype(jnp.float32) * (2.0 ** -18)
            lo2 = (r3 * (2.0 ** 27)).astype(jnp.bfloat16)
            es_ref[c, :, 0 * d:1 * d] = hi
            es_ref[c, :, 1 * d:2 * d] = mid
            es_ref[c, :, 2 * d:3 * d] = lo
            es_ref[c, :, 3 * d:4 * d] = lo2

    iota_f = jax.lax.broadcasted_iota(jnp.int32, (h, k), 1).astype(jnp.float32)
    xs = [xt_ref[:h], xt_ref[h:]]
    zqs = [jnp.zeros((h, d), jnp.float32) for _ in range(2)]
    for c in range(c_num):
        es = es_ref[c]
        for j in range(2):
            x_res = xs[j]
            x2 = jnp.sum(x_res * x_res, axis=1, keepdims=True)
            p2 = jax.lax.dot_general(
                x_res * -2.0, e_ref[c],
                (((1,), (1,)), ((), ())))
            m_run = jnp.full((h, 1), jnp.inf, jnp.float32)
            i_run = jnp.full((h, 1), float(k), jnp.float32)
            for cc in range(nch):
                lo_, hi_ = cc * _CH, (cc + 1) * _CH
                t = (x2 + p2[:, lo_:hi_]) + e2_ref[c:c + 1, lo_:hi_]
                mc = jnp.min(t, axis=1, keepdims=True)
                ic = jnp.min(jnp.where(t == mc, iota_f[:, lo_:hi_], float(k)),
                             axis=1, keepdims=True)
                first = mc < m_run
                i_run = jnp.where(first, ic, i_run)
                m_run = jnp.minimum(mc, m_run)
            indf = i_run
            oh = (iota_f == indf).astype(jnp.bfloat16)
            parts = jax.lax.dot_general(
                oh, es, (((1,), (0,)), ((), ())),
                preferred_element_type=jnp.float32)
            sel = ((parts[:, :d] + parts[:, d:2 * d] * (2.0 ** -9))
                   + parts[:, 2 * d:3 * d] * (2.0 ** -18)) \
                + parts[:, 3 * d:] * (2.0 ** -27)
            xs[j] = x_res - sel
            zqs[j] = zqs[j] + sel
            agg_ref[c, j * h:(j + 1) * h] = zqs[j]
            ind_ref[c, j * h:(j + 1) * h] = indf[:, 0].astype(jnp.int32)


def kernel(x_in, code_embeddings):
    b, d, t = x_in.shape
    c_num, k, _ = code_embeddings.shape
    nt = b * t
    xt = jnp.transpose(x_in, (0, 2, 1)).reshape(nt, d)
    grid = (pl.cdiv(nt, _TB),)
    aggs, inds = pl.pallas_call(
        _rvq_body,
        grid=grid,
        in_specs=[
            pl.BlockSpec((_TB, d), lambda i: (i, 0)),
            pl.BlockSpec((c_num, k, d), lambda i: (0, 0, 0)),
        ],
        out_specs=[
            pl.BlockSpec((c_num, _TB, d), lambda i: (0, i, 0)),
            pl.BlockSpec((c_num, _TB), lambda i: (0, i)),
        ],
        out_shape=[
            jax.ShapeDtypeStruct((c_num, nt, d), jnp.float32),
            jax.ShapeDtypeStruct((c_num, nt), jnp.int32),
        ],
        scratch_shapes=[
            pltpu.VMEM((c_num, k, 4 * d), jnp.bfloat16),
            pltpu.VMEM((c_num, k), jnp.float32),
        ],
    )(xt, code_embeddings)
    z_q_aggregated = jnp.transpose(aggs.reshape(c_num, b, t, d), (1, 0, 3, 2))
    indices = jnp.transpose(inds.reshape(c_num, b, t), (1, 2, 0))
    return z_q_aggregated, indices
